# baseline (speedup 1.0000x reference)
import functools

import jax
import jax.numpy as jnp
from jax import lax
from jax.experimental import pallas as pl
from jax.experimental.pallas import tpu as pltpu
from jax.experimental.pallas import tpu_sc as plsc

K = 8192
D = 256
N = 16 * 1024
NT = 128
COMMIT = 0.25


SEG0 = 2736
SEG1 = 5472


def _argmin_body(sx_ref, sc_ref, x_ref, cbm2_ref, idx_ref):
    x = x_ref[...]
    m2 = lax.dot_general(x.astype(jnp.bfloat16), cbm2_ref[...],
                         (((1,), (1,)), ((), ())),
                         preferred_element_type=jnp.float32)
    d = (sx_ref[...] + sc_ref[...]) + m2
    ii = lax.broadcasted_iota(jnp.int32, (NT, K), 1)

    def seg(lo, hi):
        ds = d[:, lo:hi]
        v = jnp.min(ds, axis=1, keepdims=True)
        i = jnp.min(jnp.where(ds == v, ii[:, lo:hi], K),
                    axis=1, keepdims=True)
        return v, i

    v0, i0 = seg(0, SEG0)
    v1, i1 = seg(SEG0, SEG1)
    v2, i2 = seg(SEG1, K)

    a = v0.astype(jnp.bfloat16).astype(jnp.float32)
    t1 = v1 < a
    b = jnp.where(t1, v1, a)
    j = jnp.where(t1, i1, i0)
    c = b.astype(jnp.bfloat16).astype(jnp.float32)
    idx_ref[...] = jnp.where(v2 < c, i2, j)


_argmin = pl.pallas_call(
    _argmin_body,
    grid=(N // NT,),
    in_specs=[
        pl.BlockSpec((NT, 1), lambda i: (i, 0)),
        pl.BlockSpec((1, K), lambda i: (0, 0)),
        pl.BlockSpec((NT, D), lambda i: (i, 0)),
        pl.BlockSpec((K, D), lambda i: (0, 0)),
    ],
    out_specs=pl.BlockSpec((NT, 1), lambda i: (i, 0)),
    out_shape=jax.ShapeDtypeStruct((N, 1), jnp.int32),
    compiler_params=pltpu.CompilerParams(
        dimension_semantics=("arbitrary",)),
)


_info = plsc.get_sparse_core_info()
_NC, _NS = _info.num_cores, _info.num_subcores
NW = _NC * _NS
BPW = N // NW
CH = 128
NCH = BPW // CH


@functools.partial(
    pl.kernel,
    mesh=plsc.VectorSubcoreMesh(core_axis_name="c", subcore_axis_name="s"),
    out_type=jax.ShapeDtypeStruct((N, D), jnp.float32),
    scratch_types=[
        pltpu.VMEM((NCH, CH), jnp.int32),
        pltpu.VMEM((CH, D), jnp.float32),
        pltpu.SemaphoreType.DMA,
    ],
)
def _gather(cb_hbm, idx_hbm, out_hbm, idx_v, rows_v, sem):
    wid = lax.axis_index("s") * _NC + lax.axis_index("c")
    base = wid * BPW
    pltpu.sync_copy(idx_hbm.at[wid], idx_v)
    for j in range(NCH):
        pltpu.async_copy(cb_hbm.at[idx_v.at[j]], rows_v, sem).wait()
        pltpu.sync_copy(rows_v, out_hbm.at[pl.ds(base + j * CH, CH)])


def _epi_body(x_ref, q_ref, qst_ref, acc_ref):
    x = x_ref[...]
    diff = q_ref[...] - x
    qst_ref[...] = x + diff

    @pl.when(pl.program_id(0) == 0)
    def _():
        acc_ref[0, 0] = 0.0

    acc_ref[0, 0] += jnp.sum(diff * diff)


_epilogue = pl.pallas_call(
    _epi_body,
    grid=(N // NT,),
    in_specs=[
        pl.BlockSpec((NT, D), lambda i: (i, 0)),
        pl.BlockSpec((NT, D), lambda i: (i, 0)),
    ],
    out_specs=[
        pl.BlockSpec((NT, D), lambda i: (i, 0)),
        pl.BlockSpec((1, 1), lambda i: (0, 0), memory_space=pltpu.SMEM),
    ],
    out_shape=[
        jax.ShapeDtypeStruct((N, D), jnp.float32),
        jax.ShapeDtypeStruct((1, 1), jnp.float32),
    ],
    compiler_params=pltpu.CompilerParams(
        dimension_semantics=("arbitrary",)),
)


def kernel(inputs, codebook):
    flat = inputs.reshape(-1, D)
    sx = jnp.sum(flat ** 2, axis=1, keepdims=True)
    sc = jnp.sum(codebook ** 2, axis=1).reshape(1, K)
    idx2 = _argmin(sx, sc, flat,
                   (-2.0 * codebook).astype(jnp.bfloat16))
    idx = idx2.reshape(-1)
    quantized = _gather(codebook, idx.reshape(NW, NCH, CH))
    qst, acc = _epilogue(flat, quantized)
    mean_sq = acc[0, 0] / jnp.float32(N * D)
    loss = jnp.float32(COMMIT) * mean_sq + mean_sq
    return (qst.reshape(inputs.shape), loss,
            idx.reshape(inputs.shape[:-1]))


# speedup vs baseline: 1.1874x; 1.1874x over previous
import functools

import jax
import jax.numpy as jnp
from jax import lax
from jax.experimental import pallas as pl
from jax.experimental.pallas import tpu as pltpu
from jax.experimental.pallas import tpu_sc as plsc

K = 8192
D = 256
N = 16 * 1024
NT = 128
COMMIT = 0.25


SEG0 = 2736
SEG1 = 5472


def _argmin_body(sx_ref, sc_ref, x_ref, cbm2_ref, idx_ref):
    x = x_ref[...]
    m2 = lax.dot_general(x.astype(jnp.bfloat16), cbm2_ref[...],
                         (((1,), (0,)), ((), ())),
                         preferred_element_type=jnp.float32)
    d = (sx_ref[...] + sc_ref[...]) + m2
    ii = lax.broadcasted_iota(jnp.int32, (NT, K), 1)

    def seg(lo, hi):
        ds = d[:, lo:hi]
        v = jnp.min(ds, axis=1, keepdims=True)
        i = jnp.min(jnp.where(ds == v, ii[:, lo:hi], K),
                    axis=1, keepdims=True)
        return v, i

    v0, i0 = seg(0, SEG0)
    v1, i1 = seg(SEG0, SEG1)
    v2, i2 = seg(SEG1, K)

    a = v0.astype(jnp.bfloat16).astype(jnp.float32)
    t1 = v1 < a
    b = jnp.where(t1, v1, a)
    j = jnp.where(t1, i1, i0)
    c = b.astype(jnp.bfloat16).astype(jnp.float32)
    idx_ref[...] = jnp.where(v2 < c, i2, j)


_argmin = pl.pallas_call(
    _argmin_body,
    grid=(N // NT,),
    in_specs=[
        pl.BlockSpec((NT, 1), lambda i: (i, 0)),
        pl.BlockSpec((1, K), lambda i: (0, 0)),
        pl.BlockSpec((NT, D), lambda i: (i, 0)),
        pl.BlockSpec((D, K), lambda i: (0, 0)),
    ],
    out_specs=pl.BlockSpec((NT, 1), lambda i: (i, 0)),
    out_shape=jax.ShapeDtypeStruct((N, 1), jnp.int32),
    compiler_params=pltpu.CompilerParams(
        dimension_semantics=("arbitrary",)),
)


_info = plsc.get_sparse_core_info()
_NC, _NS = _info.num_cores, _info.num_subcores
NW = _NC * _NS
BPW = N // NW
CH = 128
NCH = BPW // CH


@functools.partial(
    pl.kernel,
    mesh=plsc.VectorSubcoreMesh(core_axis_name="c", subcore_axis_name="s"),
    out_type=jax.ShapeDtypeStruct((N, D), jnp.float32),
    scratch_types=[
        pltpu.VMEM((NCH, CH), jnp.int32),
        pltpu.VMEM((CH, D), jnp.float32),
        pltpu.SemaphoreType.DMA,
    ],
)
def _gather(cb_hbm, idx_hbm, out_hbm, idx_v, rows_v, sem):
    wid = lax.axis_index("s") * _NC + lax.axis_index("c")
    base = wid * BPW
    pltpu.sync_copy(idx_hbm.at[wid], idx_v)
    for j in range(NCH):
        pltpu.async_copy(cb_hbm.at[idx_v.at[j]], rows_v, sem).wait()
        pltpu.sync_copy(rows_v, out_hbm.at[pl.ds(base + j * CH, CH)])


def _epi_body(x_ref, q_ref, qst_ref, acc_ref):
    x = x_ref[...]
    diff = q_ref[...] - x
    qst_ref[...] = x + diff

    @pl.when(pl.program_id(0) == 0)
    def _():
        acc_ref[0, 0] = 0.0

    acc_ref[0, 0] += jnp.sum(diff * diff)


_epilogue = pl.pallas_call(
    _epi_body,
    grid=(N // NT,),
    in_specs=[
        pl.BlockSpec((NT, D), lambda i: (i, 0)),
        pl.BlockSpec((NT, D), lambda i: (i, 0)),
    ],
    out_specs=[
        pl.BlockSpec((NT, D), lambda i: (i, 0)),
        pl.BlockSpec((1, 1), lambda i: (0, 0), memory_space=pltpu.SMEM),
    ],
    out_shape=[
        jax.ShapeDtypeStruct((N, D), jnp.float32),
        jax.ShapeDtypeStruct((1, 1), jnp.float32),
    ],
    compiler_params=pltpu.CompilerParams(
        dimension_semantics=("arbitrary",)),
)


def kernel(inputs, codebook):
    flat = inputs.reshape(-1, D)
    sx = jnp.sum(flat ** 2, axis=1, keepdims=True)
    sc = jnp.sum(codebook ** 2, axis=1).reshape(1, K)
    idx2 = _argmin(sx, sc, flat,
                   (-2.0 * codebook.T).astype(jnp.bfloat16))
    idx = idx2.reshape(-1)
    quantized = _gather(codebook, idx.reshape(NW, NCH, CH))
    qst, acc = _epilogue(flat, quantized)
    mean_sq = acc[0, 0] / jnp.float32(N * D)
    loss = jnp.float32(COMMIT) * mean_sq + mean_sq
    return (qst.reshape(inputs.shape), loss,
            idx.reshape(inputs.shape[:-1]))
